# baseline (speedup 1.0000x reference)
import jax
import jax.numpy as jnp
from jax import lax
from jax.experimental import pallas as pl
from jax.experimental.pallas import tpu as pltpu
from jax.experimental.pallas import tpu_sc as plsc

_ALPHA = 0.1
_N = 32768
_D = 128
_K = 256
_NC = 2
_NS = 16
_NW = _NC * _NS
_KPW = _K // _NW
_CHUNK = 192
_CHUNKB = _CHUNK + 8
_COLS = 384


def _vec_sqrt(x):
    i = lax.bitcast_convert_type(x, jnp.int32)
    i = jnp.int32(0x5F3759DF) - lax.shift_right_logical(i, 1)
    y = lax.bitcast_convert_type(i, jnp.float32)
    half = x * 0.5
    for _ in range(3):
        y = y * (1.5 - half * y * y)
    return jnp.where(x == 0.0, 0.0, x * y)


def _sc_body(emb, labels, mass, out, lab_all, mass_all, ebuf, acc, sem):
    cid = lax.axis_index("c")
    sid = lax.axis_index("s")
    wid = sid * _NC + cid
    lab0 = pl.multiple_of(wid * _KPW, _KPW)

    pltpu.sync_copy(labels, lab_all)
    pltpu.sync_copy(mass, mass_all)

    iota16 = lax.iota(jnp.int32, 16)
    zeros16 = jnp.zeros((16,), jnp.float32)
    lane_lt8 = iota16 < 8

    for r in range(_KPW):
        for j in range(_COLS // 16):
            acc[r, pl.ds(j * 16, 16)] = zeros16

    targets = jnp.full((16,), lab0, jnp.int32) + iota16
    lo_v = jnp.full((16,), -1, jnp.int32)
    hi_v = jnp.full((16,), _N, jnp.int32)
    for _ in range(16):
        mid = lax.div(lo_v + hi_v, 2)
        v = plsc.load_gather(lab_all, [mid])
        pred = v < targets
        lo_v = jnp.where(pred, mid, lo_v)
        hi_v = jnp.where(pred, hi_v, mid)
    bounds = [hi_v[r] for r in range(_KPW + 1)]
    start = bounds[0]
    end = bounds[_KPW]

    def _prep(i, carry):
        sl = pl.ds(i * 16, 16)
        mass_all[sl] = _vec_sqrt(mass_all[sl])
        return carry
    lax.fori_loop(lax.div(start, 16), lax.div(end + 15, 16), _prep, 0)

    def _tok_body(buf, l, d, st):
        rme, rs1, rsq, rm = st
        g = jnp.full((16,), d + l, jnp.int32)
        msplat = plsc.load_gather(mass_all, [g])
        rme2, rs12, rsq2 = [], [], []
        for j in range(8):
            ej = ebuf[buf, l, pl.ds(j * 16, 16)]
            rme2.append(rme[j] + ej * msplat)
            rs12.append(rs1[j] + ej)
            rsq2.append(rsq[j] + ej * ej)
        return (rme2, rs12, rsq2, rm + msplat)

    def _run(buf, l_lo, l_hi, d):
        init = ([zeros16] * 8, [zeros16] * 8, [zeros16] * 8, zeros16)
        count = jnp.maximum(l_hi - l_lo, 0)
        half = lax.div(count, 2)

        def tok2(i, st):
            l = l_lo + i * 2
            return _tok_body(buf, l + 1, d, _tok_body(buf, l, d, st))

        st = lax.fori_loop(0, half, tok2, init)

        def tok1(l, st):
            return _tok_body(buf, l, d, st)

        st = lax.fori_loop(l_lo + half * 2, l_hi, tok1, st)
        return st, count

    nch = lax.div(end - start + (_CHUNK - 1), _CHUNK)

    def _dma_base(c):
        b = start + c * _CHUNK
        d = pl.multiple_of(
            jnp.minimum(lax.div(b, 8) * 8, _N - _CHUNKB), 8)
        return b, d

    def _dma_start(c):
        _, d = _dma_base(c)
        pltpu.async_copy(
            emb.at[pl.ds(d, _CHUNKB), :], ebuf.at[jnp.bitwise_and(c, 1)],
            sem)

    @pl.when(nch > 0)
    def _():
        _dma_start(0)

    def _chunk(c, carry):
        @pl.when(c + 1 < nch)
        def _():
            _dma_start(c + 1)
        b, d = _dma_base(c)
        buf = jnp.bitwise_and(c, 1)
        pltpu.make_async_copy(
            emb.at[pl.ds(d, _CHUNKB), :], ebuf.at[buf], sem).wait()
        l_lo = b - d
        l_hi = jnp.minimum(b + _CHUNK, end) - d
        for r in range(_KPW):
            r_lo = jnp.maximum(l_lo, bounds[r] - d)
            r_hi = jnp.minimum(l_hi, bounds[r + 1] - d)
            (rme, rs1, rsq, rm), count = _run(buf, r_lo, r_hi, d)

            @pl.when(count > 0)
            def _():
                for j in range(8):
                    plsc.addupdate(acc.at[r, pl.ds(j * 16, 16)], rme[j])
                    plsc.addupdate(acc.at[r, pl.ds(_D + j * 16, 16)], rs1[j])
                sq = rsq[0]
                for j in range(1, 8):
                    sq = sq + rsq[j]
                plsc.addupdate(acc.at[r, pl.ds(272, 16)], sq)
                cntf = jnp.full((16,), count.astype(jnp.float32))
                misc = jnp.where(lane_lt8, rm, cntf)
                plsc.addupdate(acc.at[r, pl.ds(288, 16)], misc)
        return carry

    lax.fori_loop(0, nch, _chunk, 0)

    pltpu.sync_copy(acc, out.at[pl.ds(lab0, _KPW), :])


_sc_stage1 = pl.kernel(
    _sc_body,
    out_type=jax.ShapeDtypeStruct((_K, _COLS), jnp.float32),
    mesh=plsc.VectorSubcoreMesh(
        core_axis_name="c", subcore_axis_name="s",
        num_cores=_NC, num_subcores=_NS),
    compiler_params=pltpu.CompilerParams(needs_layout_passes=False),
    scratch_types=[
        pltpu.VMEM((_N,), jnp.int32),
        pltpu.VMEM((_N,), jnp.float32),
        pltpu.VMEM((2, _CHUNKB, _D), jnp.float32),
        pltpu.VMEM((_KPW, _COLS), jnp.float32),
        pltpu.SemaphoreType.DMA,
    ],
)


def _tc_body(stats_ref, size_ref, o_ref):
    s = stats_ref[...]
    sme = s[:, 0:_D]
    s1 = s[:, _D:2 * _D]
    s2 = jnp.sum(s[:, 272:288], axis=1, keepdims=True)
    sm = s[:, 288:289]
    cnt = s[:, 296:297]
    c = sme / sm
    cs1 = jnp.sum(c * s1, axis=1, keepdims=True)
    cck = jnp.sum(c * c, axis=1, keepdims=True)
    intra = (s2 - 2.0 * cs1 + cnt * cck) / cnt
    loss_intra = jnp.sum(intra) / _K

    g = lax.dot_general(c, c, (((1,), (1,)), ((), ())),
                        preferred_element_type=jnp.float32)
    ccv = jnp.sum(c * c, axis=1)
    d2 = ccv[:, None] + ccv[None, :] - 2.0 * g
    pd = jnp.sqrt(jnp.maximum(d2, 0.0))
    q = jnp.sqrt(size_ref[...])
    qq = q[:, None] * q[None, :]
    ii = lax.broadcasted_iota(jnp.int32, (_K, _K), 0)
    jj = lax.broadcasted_iota(jnp.int32, (_K, _K), 1)
    off = ii != jj
    inter = jnp.sum(jnp.where(off, qq, 0.0) / jnp.where(off, pd, 1.0))
    loss_inter = _ALPHA * inter / (_K * (_K - 1))

    row = lax.broadcasted_iota(jnp.int32, (8, 128), 0)
    lane = lax.broadcasted_iota(jnp.int32, (8, 128), 1)
    vals = jnp.where(lane == 0, loss_intra + loss_inter,
                     jnp.where(lane == 1, loss_intra,
                               jnp.where(lane == 2, loss_inter, 0.0)))
    o_ref[...] = jnp.where(row == 0, vals, 0.0)


_tc_stage2 = pl.pallas_call(
    _tc_body,
    out_shape=jax.ShapeDtypeStruct((8, 128), jnp.float32),
)


def kernel(embeddings, labels, mass, size_map):
    stats = _sc_stage1(embeddings, labels, mass)
    o = _tc_stage2(stats, size_map)
    return (o[0, 0], o[0, 1], o[0, 2])


# speedup vs baseline: 1.0160x; 1.0160x over previous
import jax
import jax.numpy as jnp
from jax import lax
from jax.experimental import pallas as pl
from jax.experimental.pallas import tpu as pltpu
from jax.experimental.pallas import tpu_sc as plsc

_ALPHA = 0.1
_N = 32768
_D = 128
_K = 256
_NC = 2
_NS = 16
_NW = _NC * _NS
_KPW = _K // _NW
_CHUNK = 248
_CHUNKB = _CHUNK + 8
_COLS = 384


def _vec_sqrt(x):
    i = lax.bitcast_convert_type(x, jnp.int32)
    i = jnp.int32(0x5F3759DF) - lax.shift_right_logical(i, 1)
    y = lax.bitcast_convert_type(i, jnp.float32)
    half = x * 0.5
    for _ in range(3):
        y = y * (1.5 - half * y * y)
    return jnp.where(x == 0.0, 0.0, x * y)


def _sc_body(emb, labels, mass, out, lab_all, mass_ch, ebuf, acc, sem):
    cid = lax.axis_index("c")
    sid = lax.axis_index("s")
    wid = sid * _NC + cid
    lab0 = pl.multiple_of(wid * _KPW, _KPW)

    pltpu.sync_copy(labels, lab_all)

    iota16 = lax.iota(jnp.int32, 16)
    zeros16 = jnp.zeros((16,), jnp.float32)
    lane_lt8 = iota16 < 8

    for r in range(_KPW):
        for j in range(_COLS // 16):
            acc[r, pl.ds(j * 16, 16)] = zeros16

    targets = jnp.full((16,), lab0, jnp.int32) + iota16
    lo_v = jnp.full((16,), -1, jnp.int32)
    hi_v = jnp.full((16,), _N, jnp.int32)
    for _ in range(16):
        mid = lax.div(lo_v + hi_v, 2)
        v = plsc.load_gather(lab_all, [mid])
        pred = v < targets
        lo_v = jnp.where(pred, mid, lo_v)
        hi_v = jnp.where(pred, hi_v, mid)
    bounds = [hi_v[r] for r in range(_KPW + 1)]
    start = bounds[0]
    end = bounds[_KPW]

    def _tok_body(buf, l, d, st):
        rme, rs1, rsq, rm = st
        g = jnp.full((16,), buf * _CHUNKB + l, jnp.int32)
        msplat = plsc.load_gather(mass_ch, [g])
        rme2, rs12, rsq2 = [], [], []
        for j in range(8):
            ej = ebuf[buf, l, pl.ds(j * 16, 16)]
            rme2.append(rme[j] + ej * msplat)
            rs12.append(rs1[j] + ej)
            rsq2.append(rsq[j] + ej * ej)
        return (rme2, rs12, rsq2, rm + msplat)

    def _run(buf, l_lo, l_hi, d):
        init = ([zeros16] * 8, [zeros16] * 8, [zeros16] * 8, zeros16)
        count = jnp.maximum(l_hi - l_lo, 0)
        half = lax.div(count, 2)

        def tok2(i, st):
            l = l_lo + i * 2
            return _tok_body(buf, l + 1, d, _tok_body(buf, l, d, st))

        st = lax.fori_loop(0, half, tok2, init)

        def tok1(l, st):
            return _tok_body(buf, l, d, st)

        st = lax.fori_loop(l_lo + half * 2, l_hi, tok1, st)
        return st, count

    nch = lax.div(end - start + (_CHUNK - 1), _CHUNK)

    def _dma_base(c):
        b = start + c * _CHUNK
        d = pl.multiple_of(
            jnp.minimum(lax.div(b, 8) * 8, _N - _CHUNKB), 8)
        return b, d

    def _dma_start(c):
        _, d = _dma_base(c)
        buf = jnp.bitwise_and(c, 1)
        pltpu.async_copy(emb.at[pl.ds(d, _CHUNKB), :], ebuf.at[buf], sem)
        pltpu.async_copy(mass.at[pl.ds(d, _CHUNKB)],
                         mass_ch.at[pl.ds(buf * _CHUNKB, _CHUNKB)], sem)

    @pl.when(nch > 0)
    def _():
        _dma_start(0)

    def _chunk(c, carry):
        @pl.when(c + 1 < nch)
        def _():
            _dma_start(c + 1)
        b, d = _dma_base(c)
        buf = jnp.bitwise_and(c, 1)
        pltpu.make_async_copy(
            emb.at[pl.ds(d, _CHUNKB), :], ebuf.at[buf], sem).wait()
        pltpu.make_async_copy(
            mass.at[pl.ds(d, _CHUNKB)],
            mass_ch.at[pl.ds(buf * _CHUNKB, _CHUNKB)], sem).wait()

        for i in range(_CHUNKB // 16):
            sl = pl.ds(buf * _CHUNKB + i * 16, 16)
            mass_ch[sl] = _vec_sqrt(mass_ch[sl])
        l_lo = b - d
        l_hi = jnp.minimum(b + _CHUNK, end) - d
        for r in range(_KPW):
            r_lo = jnp.maximum(l_lo, bounds[r] - d)
            r_hi = jnp.minimum(l_hi, bounds[r + 1] - d)
            (rme, rs1, rsq, rm), count = _run(buf, r_lo, r_hi, d)

            @pl.when(count > 0)
            def _():
                for j in range(8):
                    plsc.addupdate(acc.at[r, pl.ds(j * 16, 16)], rme[j])
                    plsc.addupdate(acc.at[r, pl.ds(_D + j * 16, 16)], rs1[j])
                sq = rsq[0]
                for j in range(1, 8):
                    sq = sq + rsq[j]
                plsc.addupdate(acc.at[r, pl.ds(272, 16)], sq)
                cntf = jnp.full((16,), count.astype(jnp.float32))
                misc = jnp.where(lane_lt8, rm, cntf)
                plsc.addupdate(acc.at[r, pl.ds(288, 16)], misc)
        return carry

    lax.fori_loop(0, nch, _chunk, 0)

    pltpu.sync_copy(acc, out.at[pl.ds(lab0, _KPW), :])


_sc_stage1 = pl.kernel(
    _sc_body,
    out_type=jax.ShapeDtypeStruct((_K, _COLS), jnp.float32),
    mesh=plsc.VectorSubcoreMesh(
        core_axis_name="c", subcore_axis_name="s",
        num_cores=_NC, num_subcores=_NS),
    compiler_params=pltpu.CompilerParams(needs_layout_passes=False),
    scratch_types=[
        pltpu.VMEM((_N,), jnp.int32),
        pltpu.VMEM((2 * _CHUNKB,), jnp.float32),
        pltpu.VMEM((2, _CHUNKB, _D), jnp.float32),
        pltpu.VMEM((_KPW, _COLS), jnp.float32),
        pltpu.SemaphoreType.DMA,
    ],
)


def _tc_body(stats_ref, size_ref, o_ref):
    s = stats_ref[...]
    sme = s[:, 0:_D]
    s1 = s[:, _D:2 * _D]
    s2 = jnp.sum(s[:, 272:288], axis=1, keepdims=True)
    sm = s[:, 288:289]
    cnt = s[:, 296:297]
    c = sme / sm
    cs1 = jnp.sum(c * s1, axis=1, keepdims=True)
    cck = jnp.sum(c * c, axis=1, keepdims=True)
    intra = (s2 - 2.0 * cs1 + cnt * cck) / cnt
    loss_intra = jnp.sum(intra) / _K

    g = lax.dot_general(c, c, (((1,), (1,)), ((), ())),
                        preferred_element_type=jnp.float32)
    ccv = jnp.sum(c * c, axis=1)
    d2 = ccv[:, None] + ccv[None, :] - 2.0 * g
    pd = jnp.sqrt(jnp.maximum(d2, 0.0))
    q = jnp.sqrt(size_ref[...])
    qq = q[:, None] * q[None, :]
    ii = lax.broadcasted_iota(jnp.int32, (_K, _K), 0)
    jj = lax.broadcasted_iota(jnp.int32, (_K, _K), 1)
    off = ii != jj
    inter = jnp.sum(jnp.where(off, qq, 0.0) / jnp.where(off, pd, 1.0))
    loss_inter = _ALPHA * inter / (_K * (_K - 1))

    row = lax.broadcasted_iota(jnp.int32, (8, 128), 0)
    lane = lax.broadcasted_iota(jnp.int32, (8, 128), 1)
    vals = jnp.where(lane == 0, loss_intra + loss_inter,
                     jnp.where(lane == 1, loss_intra,
                               jnp.where(lane == 2, loss_inter, 0.0)))
    o_ref[...] = jnp.where(row == 0, vals, 0.0)


_tc_stage2 = pl.pallas_call(
    _tc_body,
    out_shape=jax.ShapeDtypeStruct((8, 128), jnp.float32),
)


def kernel(embeddings, labels, mass, size_map):
    stats = _sc_stage1(embeddings, labels, mass)
    o = _tc_stage2(stats, size_map)
    return (o[0, 0], o[0, 1], o[0, 2])


# speedup vs baseline: 1.0355x; 1.0192x over previous
import jax
import jax.numpy as jnp
from jax import lax
from jax.experimental import pallas as pl
from jax.experimental.pallas import tpu as pltpu
from jax.experimental.pallas import tpu_sc as plsc

_ALPHA = 0.1
_N = 32768
_D = 128
_K = 256
_NC = 2
_NS = 16
_NW = _NC * _NS
_KPW = _K // _NW
_CHUNK = 344
_CHUNKB = _CHUNK + 8
_COLS = 384


def _vec_sqrt(x):
    i = lax.bitcast_convert_type(x, jnp.int32)
    i = jnp.int32(0x5F3759DF) - lax.shift_right_logical(i, 1)
    y = lax.bitcast_convert_type(i, jnp.float32)
    half = x * 0.5
    for _ in range(3):
        y = y * (1.5 - half * y * y)
    return jnp.where(x == 0.0, 0.0, x * y)


def _sc_body(emb, labels, mass, out, lab_all, mass_ch, ebuf, acc, sem):
    cid = lax.axis_index("c")
    sid = lax.axis_index("s")
    wid = sid * _NC + cid
    lab0 = pl.multiple_of(wid * _KPW, _KPW)

    pltpu.sync_copy(labels, lab_all)

    iota16 = lax.iota(jnp.int32, 16)
    zeros16 = jnp.zeros((16,), jnp.float32)
    lane_lt8 = iota16 < 8

    for r in range(_KPW):
        for j in range(_COLS // 16):
            acc[r, pl.ds(j * 16, 16)] = zeros16

    targets = jnp.full((16,), lab0, jnp.int32) + iota16
    lo_v = jnp.full((16,), -1, jnp.int32)
    hi_v = jnp.full((16,), _N, jnp.int32)
    for _ in range(16):
        mid = lax.div(lo_v + hi_v, 2)
        v = plsc.load_gather(lab_all, [mid])
        pred = v < targets
        lo_v = jnp.where(pred, mid, lo_v)
        hi_v = jnp.where(pred, hi_v, mid)
    bounds = [hi_v[r] for r in range(_KPW + 1)]
    start = bounds[0]
    end = bounds[_KPW]

    def _tok_body(buf, l, d, st):
        rme, rs1, rsq, rm = st
        g = jnp.full((16,), buf * _CHUNKB + l, jnp.int32)
        msplat = plsc.load_gather(mass_ch, [g])
        rme2, rs12, rsq2 = [], [], []
        for j in range(8):
            ej = ebuf[buf, l, pl.ds(j * 16, 16)]
            rme2.append(rme[j] + ej * msplat)
            rs12.append(rs1[j] + ej)
            rsq2.append(rsq[j] + ej * ej)
        return (rme2, rs12, rsq2, rm + msplat)

    def _run(buf, l_lo, l_hi, d):
        init = ([zeros16] * 8, [zeros16] * 8, [zeros16] * 8, zeros16)
        count = jnp.maximum(l_hi - l_lo, 0)
        half = lax.div(count, 2)

        def tok2(i, st):
            l = l_lo + i * 2
            return _tok_body(buf, l + 1, d, _tok_body(buf, l, d, st))

        st = lax.fori_loop(0, half, tok2, init)

        def tok1(l, st):
            return _tok_body(buf, l, d, st)

        st = lax.fori_loop(l_lo + half * 2, l_hi, tok1, st)
        return st, count

    nch = lax.div(end - start + (_CHUNK - 1), _CHUNK)

    def _dma_base(c):
        b = start + c * _CHUNK
        d = pl.multiple_of(
            jnp.minimum(lax.div(b, 8) * 8, _N - _CHUNKB), 8)
        return b, d

    def _dma_start(c):
        _, d = _dma_base(c)
        buf = jnp.bitwise_and(c, 1)
        pltpu.async_copy(emb.at[pl.ds(d, _CHUNKB), :], ebuf.at[buf], sem)
        pltpu.async_copy(mass.at[pl.ds(d, _CHUNKB)],
                         mass_ch.at[pl.ds(buf * _CHUNKB, _CHUNKB)], sem)

    @pl.when(nch > 0)
    def _():
        _dma_start(0)

    def _chunk(c, carry):
        @pl.when(c + 1 < nch)
        def _():
            _dma_start(c + 1)
        b, d = _dma_base(c)
        buf = jnp.bitwise_and(c, 1)
        pltpu.make_async_copy(
            emb.at[pl.ds(d, _CHUNKB), :], ebuf.at[buf], sem).wait()
        pltpu.make_async_copy(
            mass.at[pl.ds(d, _CHUNKB)],
            mass_ch.at[pl.ds(buf * _CHUNKB, _CHUNKB)], sem).wait()

        for i in range(_CHUNKB // 16):
            sl = pl.ds(buf * _CHUNKB + i * 16, 16)
            mass_ch[sl] = _vec_sqrt(mass_ch[sl])
        l_lo = b - d
        l_hi = jnp.minimum(b + _CHUNK, end) - d
        for r in range(_KPW):
            r_lo = jnp.maximum(l_lo, bounds[r] - d)
            r_hi = jnp.minimum(l_hi, bounds[r + 1] - d)
            (rme, rs1, rsq, rm), count = _run(buf, r_lo, r_hi, d)

            @pl.when(count > 0)
            def _():
                for j in range(8):
                    plsc.addupdate(acc.at[r, pl.ds(j * 16, 16)], rme[j])
                    plsc.addupdate(acc.at[r, pl.ds(_D + j * 16, 16)], rs1[j])
                sq = rsq[0]
                for j in range(1, 8):
                    sq = sq + rsq[j]
                plsc.addupdate(acc.at[r, pl.ds(272, 16)], sq)
                cntf = jnp.full((16,), count.astype(jnp.float32))
                misc = jnp.where(lane_lt8, rm, cntf)
                plsc.addupdate(acc.at[r, pl.ds(288, 16)], misc)
        return carry

    lax.fori_loop(0, nch, _chunk, 0)

    pltpu.sync_copy(acc, out.at[pl.ds(lab0, _KPW), :])


_sc_stage1 = pl.kernel(
    _sc_body,
    out_type=jax.ShapeDtypeStruct((_K, _COLS), jnp.float32),
    mesh=plsc.VectorSubcoreMesh(
        core_axis_name="c", subcore_axis_name="s",
        num_cores=_NC, num_subcores=_NS),
    compiler_params=pltpu.CompilerParams(needs_layout_passes=False),
    scratch_types=[
        pltpu.VMEM((_N,), jnp.int32),
        pltpu.VMEM((2 * _CHUNKB,), jnp.float32),
        pltpu.VMEM((2, _CHUNKB, _D), jnp.float32),
        pltpu.VMEM((_KPW, _COLS), jnp.float32),
        pltpu.SemaphoreType.DMA,
    ],
)


def _tc_body(stats_ref, size_ref, o_ref):
    s = stats_ref[...]
    sme = s[:, 0:_D]
    s1 = s[:, _D:2 * _D]
    s2 = jnp.sum(s[:, 272:288], axis=1, keepdims=True)
    sm = s[:, 288:289]
    cnt = s[:, 296:297]
    c = sme / sm
    cs1 = jnp.sum(c * s1, axis=1, keepdims=True)
    cck = jnp.sum(c * c, axis=1, keepdims=True)
    intra = (s2 - 2.0 * cs1 + cnt * cck) / cnt
    loss_intra = jnp.sum(intra) / _K

    g = lax.dot_general(c, c, (((1,), (1,)), ((), ())),
                        preferred_element_type=jnp.float32)
    ccv = jnp.sum(c * c, axis=1)
    d2 = ccv[:, None] + ccv[None, :] - 2.0 * g
    pd = jnp.sqrt(jnp.maximum(d2, 0.0))
    q = jnp.sqrt(size_ref[...])
    qq = q[:, None] * q[None, :]
    ii = lax.broadcasted_iota(jnp.int32, (_K, _K), 0)
    jj = lax.broadcasted_iota(jnp.int32, (_K, _K), 1)
    off = ii != jj
    inter = jnp.sum(jnp.where(off, qq, 0.0) / jnp.where(off, pd, 1.0))
    loss_inter = _ALPHA * inter / (_K * (_K - 1))

    row = lax.broadcasted_iota(jnp.int32, (8, 128), 0)
    lane = lax.broadcasted_iota(jnp.int32, (8, 128), 1)
    vals = jnp.where(lane == 0, loss_intra + loss_inter,
                     jnp.where(lane == 1, loss_intra,
                               jnp.where(lane == 2, loss_inter, 0.0)))
    o_ref[...] = jnp.where(row == 0, vals, 0.0)


_tc_stage2 = pl.pallas_call(
    _tc_body,
    out_shape=jax.ShapeDtypeStruct((8, 128), jnp.float32),
)


def kernel(embeddings, labels, mass, size_map):
    stats = _sc_stage1(embeddings, labels, mass)
    o = _tc_stage2(stats, size_map)
    return (o[0, 0], o[0, 1], o[0, 2])
